# baseline (speedup 1.0000x reference)
import functools

import jax
import jax.numpy as jnp
from jax.experimental import pallas as pl
from jax.experimental.pallas import tpu as pltpu
from jax.experimental.pallas import tpu_sc as plsc

VOCAB = 100000
EMB_DIM = 16
B, H, W = 256, 64, 64
HW = H * W
N = B * HW
WINDOW = 128
PACK = 128 // EMB_DIM
NP = N // PACK
WP = WINDOW // PACK


def _sc_gather(emb0, emb1, idx2):
    nc = idx2.shape[1]
    mesh = plsc.VectorSubcoreMesh(core_axis_name="c", subcore_axis_name="s")
    out_t = (jax.ShapeDtypeStruct((nc, EMB_DIM), jnp.float32),
             jax.ShapeDtypeStruct((nc, EMB_DIM), jnp.float32))

    @functools.partial(
        pl.kernel, out_type=out_t, mesh=mesh,
        scratch_types=[pltpu.SemaphoreType.DMA, pltpu.SemaphoreType.DMA],
        compiler_params=pltpu.CompilerParams(use_tc_tiling_on_sc=False))
    def k(emb0_hbm, emb1_hbm, idx_hbm, g0_hbm, g1_hbm, sem0, sem1):
        def body(i_vmem, o0_vmem, o1_vmem):
            c0 = pltpu.async_copy(emb0_hbm.at[i_vmem.at[0]], o0_vmem, sem0)
            c1 = pltpu.async_copy(emb1_hbm.at[i_vmem.at[1]], o1_vmem, sem1)
            c0.wait()
            c1.wait()

        pltpu.emit_pipeline(
            body,
            grid=(nc // WINDOW,),
            in_specs=[pl.BlockSpec((2, WINDOW), lambda i: (0, i))],
            out_specs=[pl.BlockSpec((WINDOW, EMB_DIM), lambda i: (i, 0)),
                       pl.BlockSpec((WINDOW, EMB_DIM), lambda i: (i, 0))],
            core_axis_name=("c", "s"),
            dimension_semantics=(pltpu.PARALLEL,),
        )(idx_hbm, g0_hbm, g1_hbm)

    return k(emb0, emb1, idx2)


def _tc_assemble(g0, g1, o23):
    def tc_body(g0_ref, g1_ref, o23_ref, out_ref):
        t0 = g0_ref[...].T
        t1 = g1_ref[...].T
        for o in range(PACK):
            out_ref[0, 0:EMB_DIM, o * 512:(o + 1) * 512] = (
                t0[o * EMB_DIM:(o + 1) * EMB_DIM, :])
            out_ref[0, EMB_DIM:2 * EMB_DIM, o * 512:(o + 1) * 512] = (
                t1[o * EMB_DIM:(o + 1) * EMB_DIM, :])
        out_ref[0, 2 * EMB_DIM, :] = (
            o23_ref[0, 0, 0, :].astype(jnp.float32) * (1.0 / (VOCAB - 1.0)))
        out_ref[0, 2 * EMB_DIM + 1, :] = (
            (o23_ref[1, 0, 0, :] & 1).astype(jnp.float32))

    bc = o23.shape[1]
    return pl.pallas_call(
        tc_body,
        grid=(bc,),
        in_specs=[pl.BlockSpec((HW // PACK, 128), lambda b: (b, 0)),
                  pl.BlockSpec((HW // PACK, 128), lambda b: (b, 0)),
                  pl.BlockSpec((2, 1, 1, HW), lambda b: (0, b, 0, 0))],
        out_specs=pl.BlockSpec((1, 2 * EMB_DIM + 2, HW), lambda b: (b, 0, 0)),
        out_shape=jax.ShapeDtypeStruct((bc, 2 * EMB_DIM + 2, HW), jnp.float32),
    )(g0, g1, o23)


NCHUNK = 4
BC = B // NCHUNK


def kernel(O, emb_cat0, emb_cat1):
    outs = []
    for c in range(NCHUNK):
        Oc = O[c * BC:(c + 1) * BC]
        idx2 = jnp.transpose(Oc.reshape(BC, PACK, HW // PACK, 4),
                             (3, 0, 2, 1))[0:2].reshape(2, BC * HW)
        o23 = jnp.transpose(Oc.reshape(BC, HW, 4)[..., 2:4],
                            (2, 0, 1)).reshape(2, BC, 1, HW)
        g0, g1 = _sc_gather(emb_cat0, emb_cat1, idx2)
        outs.append(_tc_assemble(g0.reshape(BC * HW // PACK, 128),
                                 g1.reshape(BC * HW // PACK, 128), o23))
    out = jnp.concatenate(outs, axis=0)
    return out.reshape(B, 2 * EMB_DIM + 2, H, W)


# speedup vs baseline: 1.2070x; 1.2070x over previous
import functools

import jax
import jax.numpy as jnp
from jax.experimental import pallas as pl
from jax.experimental.pallas import tpu as pltpu
from jax.experimental.pallas import tpu_sc as plsc

VOCAB = 100000
EMB_DIM = 16
B, H, W = 256, 64, 64
HW = H * W
N = B * HW
WINDOW = 128
PACK = 128 // EMB_DIM
NP = N // PACK
WP = WINDOW // PACK


def _sc_gather(emb0, emb1, idx2):
    nc = idx2.shape[1]
    mesh = plsc.VectorSubcoreMesh(core_axis_name="c", subcore_axis_name="s")
    out_t = (jax.ShapeDtypeStruct((nc, EMB_DIM), jnp.float32),
             jax.ShapeDtypeStruct((nc, EMB_DIM), jnp.float32))

    @functools.partial(
        pl.kernel, out_type=out_t, mesh=mesh,
        scratch_types=[pltpu.SemaphoreType.DMA, pltpu.SemaphoreType.DMA],
        compiler_params=pltpu.CompilerParams(use_tc_tiling_on_sc=False))
    def k(emb0_hbm, emb1_hbm, idx_hbm, g0_hbm, g1_hbm, sem0, sem1):
        def body(i_vmem, o0_vmem, o1_vmem):
            c0 = pltpu.async_copy(emb0_hbm.at[i_vmem.at[0]], o0_vmem, sem0)
            c1 = pltpu.async_copy(emb1_hbm.at[i_vmem.at[1]], o1_vmem, sem1)
            c0.wait()
            c1.wait()

        pltpu.emit_pipeline(
            body,
            grid=(nc // WINDOW,),
            in_specs=[pl.BlockSpec((2, WINDOW), lambda i: (0, i))],
            out_specs=[pl.BlockSpec((WINDOW, EMB_DIM), lambda i: (i, 0)),
                       pl.BlockSpec((WINDOW, EMB_DIM), lambda i: (i, 0))],
            core_axis_name=("c", "s"),
            dimension_semantics=(pltpu.PARALLEL,),
        )(idx_hbm, g0_hbm, g1_hbm)

    return k(emb0, emb1, idx2)


def _tc_assemble(g0, g1, o23):
    def tc_body(g0_ref, g1_ref, o23_ref, out_ref):
        t0 = g0_ref[...].T
        t1 = g1_ref[...].T
        for o in range(PACK):
            out_ref[0, 0:EMB_DIM, o * 512:(o + 1) * 512] = (
                t0[o * EMB_DIM:(o + 1) * EMB_DIM, :])
            out_ref[0, EMB_DIM:2 * EMB_DIM, o * 512:(o + 1) * 512] = (
                t1[o * EMB_DIM:(o + 1) * EMB_DIM, :])
        out_ref[0, 2 * EMB_DIM, :] = (
            o23_ref[0, 0, 0, :].astype(jnp.float32) * (1.0 / (VOCAB - 1.0)))
        out_ref[0, 2 * EMB_DIM + 1, :] = (
            (o23_ref[1, 0, 0, :] & 1).astype(jnp.float32))

    bc = o23.shape[1]
    return pl.pallas_call(
        tc_body,
        grid=(bc,),
        in_specs=[pl.BlockSpec((HW // PACK, 128), lambda b: (b, 0)),
                  pl.BlockSpec((HW // PACK, 128), lambda b: (b, 0)),
                  pl.BlockSpec((2, 1, 1, HW), lambda b: (0, b, 0, 0))],
        out_specs=pl.BlockSpec((1, 2 * EMB_DIM + 2, HW), lambda b: (b, 0, 0)),
        out_shape=jax.ShapeDtypeStruct((bc, 2 * EMB_DIM + 2, HW), jnp.float32),
    )(g0, g1, o23)


def kernel(O, emb_cat0, emb_cat1):
    O4 = O.reshape(B, PACK, HW // PACK, 4)
    idx2 = jnp.transpose(O4[..., 0:2], (3, 0, 2, 1)).reshape(2, N)
    o23 = jnp.transpose(O.reshape(B, HW, 4)[..., 2:4],
                        (2, 0, 1)).reshape(2, B, 1, HW)

    g0, g1 = _sc_gather(emb_cat0, emb_cat1, idx2)
    out = _tc_assemble(g0.reshape(NP, 128), g1.reshape(NP, 128), o23)
    return out.reshape(B, 2 * EMB_DIM + 2, H, W)


# speedup vs baseline: 1.3735x; 1.1379x over previous
import functools

import jax
import jax.numpy as jnp
from jax.experimental import pallas as pl
from jax.experimental.pallas import tpu as pltpu
from jax.experimental.pallas import tpu_sc as plsc

VOCAB = 100000
EMB_DIM = 16
B, H, W = 256, 64, 64
HW = H * W
N = B * HW
WINDOW = 128
PACK = 128 // EMB_DIM
NP = N // PACK
WP = WINDOW // PACK
BPG = 4


def _sc_gather(emb0, emb1, idx2):
    nc = idx2.shape[1]
    mesh = plsc.VectorSubcoreMesh(core_axis_name="c", subcore_axis_name="s")
    out_t = (jax.ShapeDtypeStruct((nc, EMB_DIM), jnp.float32),
             jax.ShapeDtypeStruct((nc, EMB_DIM), jnp.float32))

    @functools.partial(
        pl.kernel, out_type=out_t, mesh=mesh,
        scratch_types=[pltpu.SemaphoreType.DMA, pltpu.SemaphoreType.DMA],
        compiler_params=pltpu.CompilerParams(use_tc_tiling_on_sc=False))
    def k(emb0_hbm, emb1_hbm, idx_hbm, g0_hbm, g1_hbm, sem0, sem1):
        def body(i_vmem, o0_vmem, o1_vmem):
            c0 = pltpu.async_copy(emb0_hbm.at[i_vmem.at[0]], o0_vmem, sem0)
            c1 = pltpu.async_copy(emb1_hbm.at[i_vmem.at[1]], o1_vmem, sem1)
            c0.wait()
            c1.wait()

        pltpu.emit_pipeline(
            body,
            grid=(nc // WINDOW,),
            in_specs=[pl.BlockSpec((2, WINDOW), lambda i: (0, i))],
            out_specs=[pl.BlockSpec((WINDOW, EMB_DIM), lambda i: (i, 0)),
                       pl.BlockSpec((WINDOW, EMB_DIM), lambda i: (i, 0))],
            core_axis_name=("c", "s"),
            dimension_semantics=(pltpu.PARALLEL,),
        )(idx_hbm, g0_hbm, g1_hbm)

    return k(emb0, emb1, idx2)


def _tc_assemble(g0, g1, o23):
    def tc_body(g0_ref, g1_ref, o23_ref, out_ref):
        t0 = g0_ref[...].T
        t1 = g1_ref[...].T
        for sb in range(BPG):
            for o in range(PACK):
                cols = slice(sb * 512, (sb + 1) * 512)
                out_ref[sb, 0:EMB_DIM, o * 512:(o + 1) * 512] = (
                    t0[o * EMB_DIM:(o + 1) * EMB_DIM, cols])
                out_ref[sb, EMB_DIM:2 * EMB_DIM, o * 512:(o + 1) * 512] = (
                    t1[o * EMB_DIM:(o + 1) * EMB_DIM, cols])
            out_ref[sb, 2 * EMB_DIM, :] = (
                o23_ref[0, sb, 0, :].astype(jnp.float32)
                * (1.0 / (VOCAB - 1.0)))
            out_ref[sb, 2 * EMB_DIM + 1, :] = (
                (o23_ref[1, sb, 0, :] & 1).astype(jnp.float32))

    bc = o23.shape[1]
    return pl.pallas_call(
        tc_body,
        grid=(bc // BPG,),
        in_specs=[pl.BlockSpec((BPG * HW // PACK, 128), lambda b: (b, 0)),
                  pl.BlockSpec((BPG * HW // PACK, 128), lambda b: (b, 0)),
                  pl.BlockSpec((2, BPG, 1, HW), lambda b: (0, b, 0, 0))],
        out_specs=pl.BlockSpec((BPG, 2 * EMB_DIM + 2, HW),
                               lambda b: (b, 0, 0)),
        out_shape=jax.ShapeDtypeStruct((bc, 2 * EMB_DIM + 2, HW), jnp.float32),
    )(g0, g1, o23)


def kernel(O, emb_cat0, emb_cat1):
    O4 = O.reshape(B, PACK, HW // PACK, 4)
    idx2 = jnp.transpose(O4[..., 0:2], (3, 0, 2, 1)).reshape(2, N)
    o23 = jnp.transpose(O.reshape(B, HW, 4)[..., 2:4],
                        (2, 0, 1)).reshape(2, B, 1, HW)

    g0, g1 = _sc_gather(emb_cat0, emb_cat1, idx2)
    out = _tc_assemble(g0.reshape(NP, 128), g1.reshape(NP, 128), o23)
    return out.reshape(B, 2 * EMB_DIM + 2, H, W)


# speedup vs baseline: 1.3911x; 1.0128x over previous
import functools

import jax
import jax.numpy as jnp
from jax.experimental import pallas as pl
from jax.experimental.pallas import tpu as pltpu
from jax.experimental.pallas import tpu_sc as plsc

VOCAB = 100000
EMB_DIM = 16
B, H, W = 256, 64, 64
HW = H * W
N = B * HW
WINDOW = 128
PACK = 128 // EMB_DIM
NP = N // PACK
WP = WINDOW // PACK
BPG = 8


def _sc_gather(emb0, emb1, idx2):
    nc = idx2.shape[1]
    mesh = plsc.VectorSubcoreMesh(core_axis_name="c", subcore_axis_name="s")
    out_t = (jax.ShapeDtypeStruct((nc, EMB_DIM), jnp.float32),
             jax.ShapeDtypeStruct((nc, EMB_DIM), jnp.float32))

    @functools.partial(
        pl.kernel, out_type=out_t, mesh=mesh,
        scratch_types=[pltpu.SemaphoreType.DMA, pltpu.SemaphoreType.DMA],
        compiler_params=pltpu.CompilerParams(use_tc_tiling_on_sc=False))
    def k(emb0_hbm, emb1_hbm, idx_hbm, g0_hbm, g1_hbm, sem0, sem1):
        def body(i_vmem, o0_vmem, o1_vmem):
            c0 = pltpu.async_copy(emb0_hbm.at[i_vmem.at[0]], o0_vmem, sem0)
            c1 = pltpu.async_copy(emb1_hbm.at[i_vmem.at[1]], o1_vmem, sem1)
            c0.wait()
            c1.wait()

        pltpu.emit_pipeline(
            body,
            grid=(nc // WINDOW,),
            in_specs=[pl.BlockSpec((2, WINDOW), lambda i: (0, i))],
            out_specs=[pl.BlockSpec((WINDOW, EMB_DIM), lambda i: (i, 0)),
                       pl.BlockSpec((WINDOW, EMB_DIM), lambda i: (i, 0))],
            core_axis_name=("c", "s"),
            dimension_semantics=(pltpu.PARALLEL,),
        )(idx_hbm, g0_hbm, g1_hbm)

    return k(emb0, emb1, idx2)


def _tc_assemble(g0, g1, o23):
    def tc_body(g0_ref, g1_ref, o23_ref, out_ref):
        t0 = g0_ref[...].T
        t1 = g1_ref[...].T
        for sb in range(BPG):
            for o in range(PACK):
                cols = slice(sb * 512, (sb + 1) * 512)
                out_ref[sb, 0:EMB_DIM, o * 512:(o + 1) * 512] = (
                    t0[o * EMB_DIM:(o + 1) * EMB_DIM, cols])
                out_ref[sb, EMB_DIM:2 * EMB_DIM, o * 512:(o + 1) * 512] = (
                    t1[o * EMB_DIM:(o + 1) * EMB_DIM, cols])
            out_ref[sb, 2 * EMB_DIM, :] = (
                o23_ref[0, sb, 0, :].astype(jnp.float32)
                * (1.0 / (VOCAB - 1.0)))
            out_ref[sb, 2 * EMB_DIM + 1, :] = (
                (o23_ref[1, sb, 0, :] & 1).astype(jnp.float32))

    bc = o23.shape[1]
    return pl.pallas_call(
        tc_body,
        grid=(bc // BPG,),
        in_specs=[pl.BlockSpec((BPG * HW // PACK, 128), lambda b: (b, 0)),
                  pl.BlockSpec((BPG * HW // PACK, 128), lambda b: (b, 0)),
                  pl.BlockSpec((2, BPG, 1, HW), lambda b: (0, b, 0, 0))],
        out_specs=pl.BlockSpec((BPG, 2 * EMB_DIM + 2, HW),
                               lambda b: (b, 0, 0)),
        out_shape=jax.ShapeDtypeStruct((bc, 2 * EMB_DIM + 2, HW), jnp.float32),
    )(g0, g1, o23)


def kernel(O, emb_cat0, emb_cat1):
    O4 = O.reshape(B, PACK, HW // PACK, 4)
    idx2 = jnp.transpose(O4[..., 0:2], (3, 0, 2, 1)).reshape(2, N)
    o23 = jnp.transpose(O.reshape(B, HW, 4)[..., 2:4],
                        (2, 0, 1)).reshape(2, B, 1, HW)

    g0, g1 = _sc_gather(emb_cat0, emb_cat1, idx2)
    out = _tc_assemble(g0.reshape(NP, 128), g1.reshape(NP, 128), o23)
    return out.reshape(B, 2 * EMB_DIM + 2, H, W)
